# baseline (speedup 1.0000x reference)
import functools

import jax
import jax.numpy as jnp
from jax import lax
from jax.experimental import pallas as pl
from jax.experimental.pallas import tpu as pltpu
from jax.experimental.pallas import tpu_sc as plsc

N = 10000
D = 128
H = 128
E = 320000
G = 64

NC = 2
NS = 16
NW = NC * NS
C = 128
NCH = 80
EP = NW * NCH * C
NPAD = 10240
RPT = NPAD // NS

_mesh = plsc.VectorSubcoreMesh(
    core_axis_name="c", subcore_axis_name="s", num_cores=NC, num_subcores=NS)


BODY = 8
NB = NCH // BODY
NCH0 = 160
NCH1 = 2 * NCH - NCH0


@functools.partial(
    pl.kernel,
    out_type=jax.ShapeDtypeStruct((NC, NPAD, H), jnp.float32),
    mesh=_mesh,
    scratch_types=[
        pltpu.VMEM((2 * BODY, C), jnp.int32),
        pltpu.VMEM((C, H), jnp.float32),
        pltpu.VMEM((C, H), jnp.float32),
        pltpu.VMEM_SHARED((NPAD, H), jnp.float32),
        pltpu.SemaphoreType.DMA,
    ],
)
def _deg_sc(edges2_hbm, out_hbm, ei, ones_v, zero_v, deg_sh, sem):
    cid = lax.axis_index("c")
    sid = lax.axis_index("s")
    wid = cid * NS + sid

    def fill(i, _):
        for j in range(H // 16):
            ones_v[i, pl.ds(j * 16, 16)] = jnp.ones((16,), jnp.float32)
            zero_v[i, pl.ds(j * 16, 16)] = jnp.zeros((16,), jnp.float32)
        return 0
    lax.fori_loop(0, C, fill, 0)

    for j in range(RPT // C):
        pltpu.sync_copy(zero_v, deg_sh.at[pl.ds(sid * RPT + j * C, C)])
    plsc.subcore_barrier()

    def body(m, _):
        row0 = pl.multiple_of(2 * (wid * NCH + m * BODY), 8)
        pltpu.sync_copy(edges2_hbm.at[pl.ds(row0, 2 * BODY)], ei)
        ds = [pltpu.async_copy(ones_v, deg_sh.at[ei.at[2 * j + 1]], sem,
                               add=True)
              for j in range(BODY)]
        for j in range(BODY):
            ds[j].wait()
        return 0
    lax.fori_loop(0, NB, body, 0)
    plsc.subcore_barrier()

    pltpu.sync_copy(deg_sh.at[pl.ds(sid * RPT, RPT)],
                    out_hbm.at[cid, pl.ds(sid * RPT, RPT)])


@functools.partial(
    pl.kernel,
    out_type=jax.ShapeDtypeStruct((NC, NPAD, H), jnp.float32),
    mesh=_mesh,
    scratch_types=[
        pltpu.VMEM((2 * BODY, C), jnp.int32),
        pltpu.VMEM((C, H), jnp.float32),
        pltpu.VMEM((C, H), jnp.float32),
        pltpu.VMEM_SHARED((NPAD, H), jnp.float32),
        pltpu.SemaphoreType.DMA,
        pltpu.SemaphoreType.DMA,
    ],
)
def _agg_sc(edges2_hbm, y_hbm, out_hbm, ei, rows0, rows1, agg_sh, sg0, sg1):
    cid = lax.axis_index("c")
    sid = lax.axis_index("s")
    rows = (rows0, rows1)
    sg = (sg0, sg1)

    cbase = jnp.where(cid == 0, sid * NCH0, NS * NCH0 + sid * NCH1)
    nb = jnp.where(cid == 0, NCH0 // BODY, NCH1 // BODY)

    def fill_zero(i, _):
        for j in range(H // 16):
            rows0[i, pl.ds(j * 16, 16)] = jnp.zeros((16,), jnp.float32)
        return 0
    lax.fori_loop(0, C, fill_zero, 0)
    for j in range(RPT // C):
        pltpu.sync_copy(rows0, agg_sh.at[pl.ds(sid * RPT + j * C, C)])
    plsc.subcore_barrier()

    def body(m, _):
        row0 = pl.multiple_of(2 * (cbase + m * BODY), 8)
        pltpu.sync_copy(edges2_hbm.at[pl.ds(row0, 2 * BODY)], ei)

        def gather(j):
            return pltpu.async_copy(y_hbm.at[ei.at[2 * j]], rows[j % 2],
                                    sg[j % 2])

        dg = {0: gather(0), 1: gather(1)}
        for j in range(BODY):
            dg[j].wait()
            dsc = pltpu.async_copy(rows[j % 2], agg_sh.at[ei.at[2 * j + 1]],
                                   sg[j % 2], add=True)
            dsc.wait()
            if j + 2 < BODY:
                dg[j + 2] = gather(j + 2)
        return 0
    lax.fori_loop(0, nb, body, 0)
    plsc.subcore_barrier()

    pltpu.sync_copy(agg_sh.at[pl.ds(sid * RPT, RPT)],
                    out_hbm.at[cid, pl.ds(sid * RPT, RPT)])


R = 400
RP = 80


def _y1_body(deg_ref, x_ref, w_ref, y_ref, dinv_ref):
    deg = deg_ref[0, :, 0:1] + deg_ref[1, :, 0:1] + 1.0
    dinv = lax.rsqrt(deg)
    xw = jnp.dot(x_ref[...], w_ref[...], preferred_element_type=jnp.float32)
    y_ref[...] = xw * dinv
    dinv_ref[...] = dinv


def _y1_call(deg_parts, x, W1):
    return pl.pallas_call(
        _y1_body,
        grid=(N // R,),
        in_specs=[
            pl.BlockSpec((NC, R, H), lambda i: (0, i, 0)),
            pl.BlockSpec((R, D), lambda i: (i, 0)),
            pl.BlockSpec((D, H), lambda i: (0, 0)),
        ],
        out_specs=[
            pl.BlockSpec((R, H), lambda i: (i, 0)),
            pl.BlockSpec((R, 1), lambda i: (i, 0)),
        ],
        out_shape=[
            jax.ShapeDtypeStruct((N, H), jnp.float32),
            jax.ShapeDtypeStruct((N, 1), jnp.float32),
        ],
    )(deg_parts, x, W1)


def _hpre_body(agg_ref, y1_ref, dinv_ref, b1_ref, hpre_ref, stats_ref):
    hp = (agg_ref[0] + agg_ref[1] + y1_ref[...]) * dinv_ref[...] + b1_ref[...]
    hpre_ref[...] = hp

    @pl.when(pl.program_id(0) == 0)
    def _():
        stats_ref[...] = jnp.zeros_like(stats_ref)

    stats_ref[0:1, :] += jnp.sum(hp, axis=0, keepdims=True)
    stats_ref[1:2, :] += jnp.sum(hp * hp, axis=0, keepdims=True)


def _hpre_call(agg1, y1, dinv, b1):
    return pl.pallas_call(
        _hpre_body,
        grid=(N // R,),
        in_specs=[
            pl.BlockSpec((NC, R, H), lambda i: (0, i, 0)),
            pl.BlockSpec((R, H), lambda i: (i, 0)),
            pl.BlockSpec((R, 1), lambda i: (i, 0)),
            pl.BlockSpec((1, H), lambda i: (0, 0)),
        ],
        out_specs=[
            pl.BlockSpec((R, H), lambda i: (i, 0)),
            pl.BlockSpec((2, H), lambda i: (0, 0)),
        ],
        out_shape=[
            jax.ShapeDtypeStruct((N, H), jnp.float32),
            jax.ShapeDtypeStruct((2, H), jnp.float32),
        ],
    )(agg1, y1, dinv, b1)


_SELU_SCALE = 1.0507009873554805
_SELU_ALPHA = 1.6732632423543772


def _y2_body(hpre_ref, stats_ref, bnw_ref, bnb_ref, w2_ref, dinv_ref, y2_ref):
    m = stats_ref[0:1, :] * (1.0 / N)
    v = stats_ref[1:2, :] * (1.0 / N) - m * m
    hn = (hpre_ref[...] - m) * lax.rsqrt(v + 1e-5) * bnw_ref[...] + bnb_ref[...]
    act = _SELU_SCALE * jnp.where(hn > 0, hn, _SELU_ALPHA * (jnp.exp(hn) - 1.0))
    y2_ref[...] = jnp.dot(act, w2_ref[...],
                          preferred_element_type=jnp.float32) * dinv_ref[...]


def _y2_call(hpre, stats, bn_w, bn_b, W2, dinv):
    return pl.pallas_call(
        _y2_body,
        grid=(N // R,),
        in_specs=[
            pl.BlockSpec((R, H), lambda i: (i, 0)),
            pl.BlockSpec((2, H), lambda i: (0, 0)),
            pl.BlockSpec((1, H), lambda i: (0, 0)),
            pl.BlockSpec((1, H), lambda i: (0, 0)),
            pl.BlockSpec((H, H), lambda i: (0, 0)),
            pl.BlockSpec((R, 1), lambda i: (i, 0)),
        ],
        out_specs=pl.BlockSpec((R, H), lambda i: (i, 0)),
        out_shape=jax.ShapeDtypeStruct((N, H), jnp.float32),
    )(hpre, stats, bn_w, bn_b, W2, dinv)


def _pool_body(agg_ref, y2_ref, dinv_ref, b2_ref, bidx_ref,
               s_ref, cnt_ref, mx_ref):
    h2 = (agg_ref[0] + agg_ref[1] + y2_ref[...]) * dinv_ref[...] + b2_ref[...]
    cols = lax.broadcasted_iota(jnp.int32, (RP, G), 1)
    mask = bidx_ref[...] == cols
    o = mask.astype(jnp.float32)

    @pl.when(pl.program_id(0) == 0)
    def _():
        s_ref[...] = jnp.zeros_like(s_ref)
        cnt_ref[...] = jnp.zeros_like(cnt_ref)
        mx_ref[...] = jnp.full_like(mx_ref, -jnp.inf)

    dims = (((0,), (0,)), ((), ()))
    s_ref[...] += lax.dot_general(o, h2, dims,
                                  preferred_element_type=jnp.float32)
    cnt_ref[...] += lax.dot_general(o, jnp.ones_like(h2), dims,
                                    preferred_element_type=jnp.float32)
    rows = []
    for g in range(G):
        sel = jnp.where(mask[:, g:g + 1], h2, -jnp.inf)
        rows.append(jnp.max(sel, axis=0, keepdims=True))
    t = jnp.concatenate(rows, axis=0)
    mx_ref[...] = jnp.maximum(mx_ref[...], t)


def _pool_call(agg2, y2, dinv, b2, bidx):
    return pl.pallas_call(
        _pool_body,
        grid=(N // RP,),
        in_specs=[
            pl.BlockSpec((NC, RP, H), lambda i: (0, i, 0)),
            pl.BlockSpec((RP, H), lambda i: (i, 0)),
            pl.BlockSpec((RP, 1), lambda i: (i, 0)),
            pl.BlockSpec((1, H), lambda i: (0, 0)),
            pl.BlockSpec((RP, 1), lambda i: (i, 0)),
        ],
        out_specs=[
            pl.BlockSpec((G, H), lambda i: (0, 0)),
            pl.BlockSpec((G, H), lambda i: (0, 0)),
            pl.BlockSpec((G, H), lambda i: (0, 0)),
        ],
        out_shape=[
            jax.ShapeDtypeStruct((G, H), jnp.float32),
            jax.ShapeDtypeStruct((G, H), jnp.float32),
            jax.ShapeDtypeStruct((G, H), jnp.float32),
        ],
    )(agg2, y2, dinv, b2, bidx)


def _final_body(s_ref, cnt_ref, mx_ref, wp_ref, bp_ref, out_ref):
    s = s_ref[...]
    cnt = cnt_ref[...]
    mean = s / jnp.maximum(cnt, 1.0)
    mx = jnp.where(cnt > 0, mx_ref[...], 0.0)
    out = jnp.dot(s, wp_ref[0:H, :], preferred_element_type=jnp.float32)
    out += jnp.dot(mean, wp_ref[H:2 * H, :], preferred_element_type=jnp.float32)
    out += jnp.dot(mx, wp_ref[2 * H:3 * H, :], preferred_element_type=jnp.float32)
    out_ref[...] = out + bp_ref[...]


def _final_call(s, cnt, mx, Wp, bp):
    return pl.pallas_call(
        _final_body,
        out_shape=jax.ShapeDtypeStruct((G, H), jnp.float32),
    )(s, cnt, mx, Wp, bp)


def kernel(x, edge_index, batch_index, W1, b1, bn_w, bn_b, W2, b2, Wp, bp):
    pad = EP - E
    src = jnp.concatenate([edge_index[0], jnp.zeros((pad,), jnp.int32)])
    pad_dst = N + jnp.arange(pad, dtype=jnp.int32) % (NPAD - N)
    dst = jnp.concatenate([edge_index[1], pad_dst])
    edges2 = jnp.stack([src.reshape(NW * NCH, C),
                        dst.reshape(NW * NCH, C)],
                       axis=1).reshape(2 * NW * NCH, C)
    deg_parts = _deg_sc(edges2)
    y1, dinv = _y1_call(deg_parts, x, W1)
    agg1 = _agg_sc(edges2, y1)
    hpre, stats = _hpre_call(agg1, y1, dinv, b1.reshape(1, H))
    y2 = _y2_call(hpre, stats, bn_w.reshape(1, H), bn_b.reshape(1, H), W2, dinv)
    agg2 = _agg_sc(edges2, y2)
    s, cnt, mx = _pool_call(agg2, y2, dinv, b2.reshape(1, H),
                            batch_index.reshape(N, 1))
    return _final_call(s, cnt, mx, Wp, bp.reshape(1, H))


# speedup vs baseline: 1.0906x; 1.0906x over previous
import functools

import jax
import jax.numpy as jnp
from jax import lax
from jax.experimental import pallas as pl
from jax.experimental.pallas import tpu as pltpu
from jax.experimental.pallas import tpu_sc as plsc

N = 10000
D = 128
H = 128
E = 320000
G = 64

NC = 2
NS = 16
NW = NC * NS
C = 128
NCH = 80
EP = NW * NCH * C
NPAD = 10240
RPT = NPAD // NS

_mesh = plsc.VectorSubcoreMesh(
    core_axis_name="c", subcore_axis_name="s", num_cores=NC, num_subcores=NS)


BODY = 8
NB = NCH // BODY
NCH0 = 80
NCH1 = 2 * NCH - NCH0


@functools.partial(
    pl.kernel,
    out_type=jax.ShapeDtypeStruct((NC, NPAD, H), jnp.float32),
    mesh=_mesh,
    scratch_types=[
        pltpu.VMEM((2 * BODY, C), jnp.int32),
        pltpu.VMEM((C, H), jnp.float32),
        pltpu.VMEM((C, H), jnp.float32),
        pltpu.VMEM_SHARED((NPAD, H), jnp.float32),
        pltpu.SemaphoreType.DMA,
    ],
)
def _deg_sc(edges2_hbm, out_hbm, ei, ones_v, zero_v, deg_sh, sem):
    cid = lax.axis_index("c")
    sid = lax.axis_index("s")
    wid = cid * NS + sid

    def fill(i, _):
        for j in range(H // 16):
            ones_v[i, pl.ds(j * 16, 16)] = jnp.ones((16,), jnp.float32)
            zero_v[i, pl.ds(j * 16, 16)] = jnp.zeros((16,), jnp.float32)
        return 0
    lax.fori_loop(0, C, fill, 0)

    for j in range(RPT // C):
        pltpu.sync_copy(zero_v, deg_sh.at[pl.ds(sid * RPT + j * C, C)])
    plsc.subcore_barrier()

    def body(m, _):
        row0 = pl.multiple_of(2 * (wid * NCH + m * BODY), 8)
        pltpu.sync_copy(edges2_hbm.at[pl.ds(row0, 2 * BODY)], ei)
        ds = [pltpu.async_copy(ones_v, deg_sh.at[ei.at[2 * j + 1]], sem,
                               add=True)
              for j in range(BODY)]
        for j in range(BODY):
            ds[j].wait()
        return 0
    lax.fori_loop(0, NB, body, 0)
    plsc.subcore_barrier()

    pltpu.sync_copy(deg_sh.at[pl.ds(sid * RPT, RPT)],
                    out_hbm.at[cid, pl.ds(sid * RPT, RPT)])


@functools.partial(
    pl.kernel,
    out_type=jax.ShapeDtypeStruct((NC, NPAD, H), jnp.float32),
    mesh=_mesh,
    scratch_types=[
        pltpu.VMEM((2 * BODY, C), jnp.int32),
        pltpu.VMEM((C, H), jnp.float32),
        pltpu.VMEM((C, H), jnp.float32),
        pltpu.VMEM_SHARED((NPAD, H), jnp.float32),
        pltpu.SemaphoreType.DMA,
        pltpu.SemaphoreType.DMA,
    ],
)
def _agg_sc(edges2_hbm, y_hbm, out_hbm, ei, rows0, rows1, agg_sh, sg0, sg1):
    cid = lax.axis_index("c")
    sid = lax.axis_index("s")
    rows = (rows0, rows1)
    sg = (sg0, sg1)

    cbase = jnp.where(cid == 0, sid * NCH0, NS * NCH0 + sid * NCH1)
    nb = jnp.where(cid == 0, NCH0 // BODY, NCH1 // BODY)

    def fill_zero(i, _):
        for j in range(H // 16):
            rows0[i, pl.ds(j * 16, 16)] = jnp.zeros((16,), jnp.float32)
        return 0
    lax.fori_loop(0, C, fill_zero, 0)
    for j in range(RPT // C):
        pltpu.sync_copy(rows0, agg_sh.at[pl.ds(sid * RPT + j * C, C)])
    plsc.subcore_barrier()

    def body(m, _):
        row0 = pl.multiple_of(2 * (cbase + m * BODY), 8)
        pltpu.sync_copy(edges2_hbm.at[pl.ds(row0, 2 * BODY)], ei)

        def gather(j):
            return pltpu.async_copy(y_hbm.at[ei.at[2 * j]], rows[j % 2],
                                    sg[j % 2])

        for j in range(BODY):
            gather(j).wait()
            pltpu.sync_copy(rows[j % 2], agg_sh.at[ei.at[2 * j + 1]],
                            add=True)
        return 0
    lax.fori_loop(0, nb, body, 0)
    plsc.subcore_barrier()

    pltpu.sync_copy(agg_sh.at[pl.ds(sid * RPT, RPT)],
                    out_hbm.at[cid, pl.ds(sid * RPT, RPT)])


R = 400
RP = 80


def _y1_body(deg_ref, x_ref, w_ref, y_ref, dinv_ref):
    deg = deg_ref[0, :, 0:1] + deg_ref[1, :, 0:1] + 1.0
    dinv = lax.rsqrt(deg)
    xw = jnp.dot(x_ref[...], w_ref[...], preferred_element_type=jnp.float32)
    y_ref[...] = xw * dinv
    dinv_ref[...] = dinv


def _y1_call(deg_parts, x, W1):
    return pl.pallas_call(
        _y1_body,
        grid=(N // R,),
        in_specs=[
            pl.BlockSpec((NC, R, H), lambda i: (0, i, 0)),
            pl.BlockSpec((R, D), lambda i: (i, 0)),
            pl.BlockSpec((D, H), lambda i: (0, 0)),
        ],
        out_specs=[
            pl.BlockSpec((R, H), lambda i: (i, 0)),
            pl.BlockSpec((R, 1), lambda i: (i, 0)),
        ],
        out_shape=[
            jax.ShapeDtypeStruct((N, H), jnp.float32),
            jax.ShapeDtypeStruct((N, 1), jnp.float32),
        ],
    )(deg_parts, x, W1)


def _hpre_body(agg_ref, y1_ref, dinv_ref, b1_ref, hpre_ref, stats_ref):
    hp = (agg_ref[0] + agg_ref[1] + y1_ref[...]) * dinv_ref[...] + b1_ref[...]
    hpre_ref[...] = hp

    @pl.when(pl.program_id(0) == 0)
    def _():
        stats_ref[...] = jnp.zeros_like(stats_ref)

    stats_ref[0:1, :] += jnp.sum(hp, axis=0, keepdims=True)
    stats_ref[1:2, :] += jnp.sum(hp * hp, axis=0, keepdims=True)


def _hpre_call(agg1, y1, dinv, b1):
    return pl.pallas_call(
        _hpre_body,
        grid=(N // R,),
        in_specs=[
            pl.BlockSpec((NC, R, H), lambda i: (0, i, 0)),
            pl.BlockSpec((R, H), lambda i: (i, 0)),
            pl.BlockSpec((R, 1), lambda i: (i, 0)),
            pl.BlockSpec((1, H), lambda i: (0, 0)),
        ],
        out_specs=[
            pl.BlockSpec((R, H), lambda i: (i, 0)),
            pl.BlockSpec((2, H), lambda i: (0, 0)),
        ],
        out_shape=[
            jax.ShapeDtypeStruct((N, H), jnp.float32),
            jax.ShapeDtypeStruct((2, H), jnp.float32),
        ],
    )(agg1, y1, dinv, b1)


_SELU_SCALE = 1.0507009873554805
_SELU_ALPHA = 1.6732632423543772


def _y2_body(hpre_ref, stats_ref, bnw_ref, bnb_ref, w2_ref, dinv_ref, y2_ref):
    m = stats_ref[0:1, :] * (1.0 / N)
    v = stats_ref[1:2, :] * (1.0 / N) - m * m
    hn = (hpre_ref[...] - m) * lax.rsqrt(v + 1e-5) * bnw_ref[...] + bnb_ref[...]
    act = _SELU_SCALE * jnp.where(hn > 0, hn, _SELU_ALPHA * (jnp.exp(hn) - 1.0))
    y2_ref[...] = jnp.dot(act, w2_ref[...],
                          preferred_element_type=jnp.float32) * dinv_ref[...]


def _y2_call(hpre, stats, bn_w, bn_b, W2, dinv):
    return pl.pallas_call(
        _y2_body,
        grid=(N // R,),
        in_specs=[
            pl.BlockSpec((R, H), lambda i: (i, 0)),
            pl.BlockSpec((2, H), lambda i: (0, 0)),
            pl.BlockSpec((1, H), lambda i: (0, 0)),
            pl.BlockSpec((1, H), lambda i: (0, 0)),
            pl.BlockSpec((H, H), lambda i: (0, 0)),
            pl.BlockSpec((R, 1), lambda i: (i, 0)),
        ],
        out_specs=pl.BlockSpec((R, H), lambda i: (i, 0)),
        out_shape=jax.ShapeDtypeStruct((N, H), jnp.float32),
    )(hpre, stats, bn_w, bn_b, W2, dinv)


def _pool_body(agg_ref, y2_ref, dinv_ref, b2_ref, bidx_ref,
               s_ref, cnt_ref, mx_ref):
    h2 = (agg_ref[0] + agg_ref[1] + y2_ref[...]) * dinv_ref[...] + b2_ref[...]
    cols = lax.broadcasted_iota(jnp.int32, (RP, G), 1)
    mask = bidx_ref[...] == cols
    o = mask.astype(jnp.float32)

    @pl.when(pl.program_id(0) == 0)
    def _():
        s_ref[...] = jnp.zeros_like(s_ref)
        cnt_ref[...] = jnp.zeros_like(cnt_ref)
        mx_ref[...] = jnp.full_like(mx_ref, -jnp.inf)

    dims = (((0,), (0,)), ((), ()))
    s_ref[...] += lax.dot_general(o, h2, dims,
                                  preferred_element_type=jnp.float32)
    cnt_ref[...] += lax.dot_general(o, jnp.ones_like(h2), dims,
                                    preferred_element_type=jnp.float32)
    rows = []
    for g in range(G):
        sel = jnp.where(mask[:, g:g + 1], h2, -jnp.inf)
        rows.append(jnp.max(sel, axis=0, keepdims=True))
    t = jnp.concatenate(rows, axis=0)
    mx_ref[...] = jnp.maximum(mx_ref[...], t)


def _pool_call(agg2, y2, dinv, b2, bidx):
    return pl.pallas_call(
        _pool_body,
        grid=(N // RP,),
        in_specs=[
            pl.BlockSpec((NC, RP, H), lambda i: (0, i, 0)),
            pl.BlockSpec((RP, H), lambda i: (i, 0)),
            pl.BlockSpec((RP, 1), lambda i: (i, 0)),
            pl.BlockSpec((1, H), lambda i: (0, 0)),
            pl.BlockSpec((RP, 1), lambda i: (i, 0)),
        ],
        out_specs=[
            pl.BlockSpec((G, H), lambda i: (0, 0)),
            pl.BlockSpec((G, H), lambda i: (0, 0)),
            pl.BlockSpec((G, H), lambda i: (0, 0)),
        ],
        out_shape=[
            jax.ShapeDtypeStruct((G, H), jnp.float32),
            jax.ShapeDtypeStruct((G, H), jnp.float32),
            jax.ShapeDtypeStruct((G, H), jnp.float32),
        ],
    )(agg2, y2, dinv, b2, bidx)


def _final_body(s_ref, cnt_ref, mx_ref, wp_ref, bp_ref, out_ref):
    s = s_ref[...]
    cnt = cnt_ref[...]
    mean = s / jnp.maximum(cnt, 1.0)
    mx = jnp.where(cnt > 0, mx_ref[...], 0.0)
    out = jnp.dot(s, wp_ref[0:H, :], preferred_element_type=jnp.float32)
    out += jnp.dot(mean, wp_ref[H:2 * H, :], preferred_element_type=jnp.float32)
    out += jnp.dot(mx, wp_ref[2 * H:3 * H, :], preferred_element_type=jnp.float32)
    out_ref[...] = out + bp_ref[...]


def _final_call(s, cnt, mx, Wp, bp):
    return pl.pallas_call(
        _final_body,
        out_shape=jax.ShapeDtypeStruct((G, H), jnp.float32),
    )(s, cnt, mx, Wp, bp)


def kernel(x, edge_index, batch_index, W1, b1, bn_w, bn_b, W2, b2, Wp, bp):
    pad = EP - E
    src = jnp.concatenate([edge_index[0], jnp.zeros((pad,), jnp.int32)])
    pad_dst = N + jnp.arange(pad, dtype=jnp.int32) % (NPAD - N)
    dst = jnp.concatenate([edge_index[1], pad_dst])
    edges2 = jnp.stack([src.reshape(NW * NCH, C),
                        dst.reshape(NW * NCH, C)],
                       axis=1).reshape(2 * NW * NCH, C)
    deg_parts = _deg_sc(edges2)
    y1, dinv = _y1_call(deg_parts, x, W1)
    agg1 = _agg_sc(edges2, y1)
    hpre, stats = _hpre_call(agg1, y1, dinv, b1.reshape(1, H))
    y2 = _y2_call(hpre, stats, bn_w.reshape(1, H), bn_b.reshape(1, H), W2, dinv)
    agg2 = _agg_sc(edges2, y2)
    s, cnt, mx = _pool_call(agg2, y2, dinv, b2.reshape(1, H),
                            batch_index.reshape(N, 1))
    return _final_call(s, cnt, mx, Wp, bp.reshape(1, H))


# speedup vs baseline: 1.4816x; 1.3585x over previous
import functools

import jax
import jax.numpy as jnp
from jax import lax
from jax.experimental import pallas as pl
from jax.experimental.pallas import tpu as pltpu
from jax.experimental.pallas import tpu_sc as plsc

N = 10000
D = 128
H = 128
E = 320000
G = 64

NC = 2
NS = 16
NW = NC * NS
C = 128
NCH = 80
EP = NW * NCH * C
NPAD = 10240
RPT = NPAD // NS

_mesh = plsc.VectorSubcoreMesh(
    core_axis_name="c", subcore_axis_name="s", num_cores=NC, num_subcores=NS)


BODY = 8
NB = NCH // BODY
NCH0 = 80
NCH1 = 2 * NCH - NCH0


@functools.partial(
    pl.kernel,
    out_type=jax.ShapeDtypeStruct((NC, NPAD, H), jnp.float32),
    mesh=_mesh,
    scratch_types=[
        pltpu.VMEM((2 * BODY, C), jnp.int32),
        pltpu.VMEM((C, H), jnp.float32),
        pltpu.VMEM((C, H), jnp.float32),
        pltpu.VMEM_SHARED((NPAD, H), jnp.float32),
        pltpu.SemaphoreType.DMA,
    ],
)
def _deg_sc(edges2_hbm, out_hbm, ei, ones_v, zero_v, deg_sh, sem):
    cid = lax.axis_index("c")
    sid = lax.axis_index("s")
    wid = cid * NS + sid

    def fill(i, _):
        for j in range(H // 16):
            ones_v[i, pl.ds(j * 16, 16)] = jnp.ones((16,), jnp.float32)
            zero_v[i, pl.ds(j * 16, 16)] = jnp.zeros((16,), jnp.float32)
        return 0
    lax.fori_loop(0, C, fill, 0)

    for j in range(RPT // C):
        pltpu.sync_copy(zero_v, deg_sh.at[pl.ds(sid * RPT + j * C, C)])
    plsc.subcore_barrier()

    def body(m, _):
        row0 = pl.multiple_of(2 * (wid * NCH + m * BODY), 8)
        pltpu.sync_copy(edges2_hbm.at[pl.ds(row0, 2 * BODY)], ei)
        ds = [pltpu.async_copy(ones_v, deg_sh.at[ei.at[2 * j + 1]], sem,
                               add=True)
              for j in range(BODY)]
        for j in range(BODY):
            ds[j].wait()
        return 0
    lax.fori_loop(0, NB, body, 0)
    plsc.subcore_barrier()

    pltpu.sync_copy(deg_sh.at[pl.ds(sid * RPT, RPT)],
                    out_hbm.at[cid, pl.ds(sid * RPT, RPT)])


CA = 80
NCHA = E // (NW * CA)


@functools.partial(
    pl.kernel,
    out_type=jax.ShapeDtypeStruct((NC, NPAD, H), jnp.float32),
    mesh=_mesh,
    scratch_types=[
        pltpu.VMEM((CA,), jnp.int32),
        pltpu.VMEM((CA,), jnp.int32),
        pltpu.VMEM((CA, H), jnp.float32),
        pltpu.VMEM((128, H), jnp.float32),
        pltpu.VMEM_SHARED((NPAD, H), jnp.float32),
        pltpu.SemaphoreType.DMA,
    ],
)
def _agg_sc(src_hbm, dst_hbm, y_hbm, out_hbm, src_v, dst_v, rows_v, zero_v,
            agg_sh, sem):
    cid = lax.axis_index("c")
    sid = lax.axis_index("s")

    def fill_zero(i, _):
        for j in range(H // 16):
            zero_v[i, pl.ds(j * 16, 16)] = jnp.zeros((16,), jnp.float32)
        return 0
    lax.fori_loop(0, 128, fill_zero, 0)

    for j in range(RPT // 128):
        pltpu.sync_copy(zero_v, agg_sh.at[pl.ds(sid * RPT + j * 128, 128)])
    plsc.subcore_barrier()

    base = (cid * NS + sid) * (NCHA * CA)

    def body(i, _):
        off = pl.multiple_of(base + i * CA, 8)
        pltpu.sync_copy(src_hbm.at[pl.ds(off, CA)], src_v)
        pltpu.sync_copy(dst_hbm.at[pl.ds(off, CA)], dst_v)
        pltpu.async_copy(y_hbm.at[src_v], rows_v, sem).wait()
        pltpu.sync_copy(rows_v, agg_sh.at[dst_v], add=True)
        return 0
    lax.fori_loop(0, NCHA, body, 0)
    plsc.subcore_barrier()

    pltpu.sync_copy(agg_sh.at[pl.ds(sid * RPT, RPT)],
                    out_hbm.at[cid, pl.ds(sid * RPT, RPT)])


R = 400
RP = 80


def _y1_body(deg_ref, x_ref, w_ref, y_ref, dinv_ref):
    deg = deg_ref[0, :, 0:1] + deg_ref[1, :, 0:1] + 1.0
    dinv = lax.rsqrt(deg)
    xw = jnp.dot(x_ref[...], w_ref[...], preferred_element_type=jnp.float32)
    y_ref[...] = xw * dinv
    dinv_ref[...] = dinv


def _y1_call(deg_parts, x, W1):
    return pl.pallas_call(
        _y1_body,
        grid=(N // R,),
        in_specs=[
            pl.BlockSpec((NC, R, H), lambda i: (0, i, 0)),
            pl.BlockSpec((R, D), lambda i: (i, 0)),
            pl.BlockSpec((D, H), lambda i: (0, 0)),
        ],
        out_specs=[
            pl.BlockSpec((R, H), lambda i: (i, 0)),
            pl.BlockSpec((R, 1), lambda i: (i, 0)),
        ],
        out_shape=[
            jax.ShapeDtypeStruct((N, H), jnp.float32),
            jax.ShapeDtypeStruct((N, 1), jnp.float32),
        ],
    )(deg_parts, x, W1)


def _hpre_body(agg_ref, y1_ref, dinv_ref, b1_ref, hpre_ref, stats_ref):
    hp = (agg_ref[0] + agg_ref[1] + y1_ref[...]) * dinv_ref[...] + b1_ref[...]
    hpre_ref[...] = hp

    @pl.when(pl.program_id(0) == 0)
    def _():
        stats_ref[...] = jnp.zeros_like(stats_ref)

    stats_ref[0:1, :] += jnp.sum(hp, axis=0, keepdims=True)
    stats_ref[1:2, :] += jnp.sum(hp * hp, axis=0, keepdims=True)


def _hpre_call(agg1, y1, dinv, b1):
    return pl.pallas_call(
        _hpre_body,
        grid=(N // R,),
        in_specs=[
            pl.BlockSpec((NC, R, H), lambda i: (0, i, 0)),
            pl.BlockSpec((R, H), lambda i: (i, 0)),
            pl.BlockSpec((R, 1), lambda i: (i, 0)),
            pl.BlockSpec((1, H), lambda i: (0, 0)),
        ],
        out_specs=[
            pl.BlockSpec((R, H), lambda i: (i, 0)),
            pl.BlockSpec((2, H), lambda i: (0, 0)),
        ],
        out_shape=[
            jax.ShapeDtypeStruct((N, H), jnp.float32),
            jax.ShapeDtypeStruct((2, H), jnp.float32),
        ],
    )(agg1, y1, dinv, b1)


_SELU_SCALE = 1.0507009873554805
_SELU_ALPHA = 1.6732632423543772


def _y2_body(hpre_ref, stats_ref, bnw_ref, bnb_ref, w2_ref, dinv_ref, y2_ref):
    m = stats_ref[0:1, :] * (1.0 / N)
    v = stats_ref[1:2, :] * (1.0 / N) - m * m
    hn = (hpre_ref[...] - m) * lax.rsqrt(v + 1e-5) * bnw_ref[...] + bnb_ref[...]
    act = _SELU_SCALE * jnp.where(hn > 0, hn, _SELU_ALPHA * (jnp.exp(hn) - 1.0))
    y2_ref[...] = jnp.dot(act, w2_ref[...],
                          preferred_element_type=jnp.float32) * dinv_ref[...]


def _y2_call(hpre, stats, bn_w, bn_b, W2, dinv):
    return pl.pallas_call(
        _y2_body,
        grid=(N // R,),
        in_specs=[
            pl.BlockSpec((R, H), lambda i: (i, 0)),
            pl.BlockSpec((2, H), lambda i: (0, 0)),
            pl.BlockSpec((1, H), lambda i: (0, 0)),
            pl.BlockSpec((1, H), lambda i: (0, 0)),
            pl.BlockSpec((H, H), lambda i: (0, 0)),
            pl.BlockSpec((R, 1), lambda i: (i, 0)),
        ],
        out_specs=pl.BlockSpec((R, H), lambda i: (i, 0)),
        out_shape=jax.ShapeDtypeStruct((N, H), jnp.float32),
    )(hpre, stats, bn_w, bn_b, W2, dinv)


def _pool_body(agg_ref, y2_ref, dinv_ref, b2_ref, bidx_ref,
               s_ref, cnt_ref, mx_ref):
    h2 = (agg_ref[0] + agg_ref[1] + y2_ref[...]) * dinv_ref[...] + b2_ref[...]
    cols = lax.broadcasted_iota(jnp.int32, (RP, G), 1)
    mask = bidx_ref[...] == cols
    o = mask.astype(jnp.float32)

    @pl.when(pl.program_id(0) == 0)
    def _():
        s_ref[...] = jnp.zeros_like(s_ref)
        cnt_ref[...] = jnp.zeros_like(cnt_ref)
        mx_ref[...] = jnp.full_like(mx_ref, -jnp.inf)

    dims = (((0,), (0,)), ((), ()))
    s_ref[...] += lax.dot_general(o, h2, dims,
                                  preferred_element_type=jnp.float32)
    cnt_ref[...] += lax.dot_general(o, jnp.ones_like(h2), dims,
                                    preferred_element_type=jnp.float32)
    rows = []
    for g in range(G):
        sel = jnp.where(mask[:, g:g + 1], h2, -jnp.inf)
        rows.append(jnp.max(sel, axis=0, keepdims=True))
    t = jnp.concatenate(rows, axis=0)
    mx_ref[...] = jnp.maximum(mx_ref[...], t)


def _pool_call(agg2, y2, dinv, b2, bidx):
    return pl.pallas_call(
        _pool_body,
        grid=(N // RP,),
        in_specs=[
            pl.BlockSpec((NC, RP, H), lambda i: (0, i, 0)),
            pl.BlockSpec((RP, H), lambda i: (i, 0)),
            pl.BlockSpec((RP, 1), lambda i: (i, 0)),
            pl.BlockSpec((1, H), lambda i: (0, 0)),
            pl.BlockSpec((RP, 1), lambda i: (i, 0)),
        ],
        out_specs=[
            pl.BlockSpec((G, H), lambda i: (0, 0)),
            pl.BlockSpec((G, H), lambda i: (0, 0)),
            pl.BlockSpec((G, H), lambda i: (0, 0)),
        ],
        out_shape=[
            jax.ShapeDtypeStruct((G, H), jnp.float32),
            jax.ShapeDtypeStruct((G, H), jnp.float32),
            jax.ShapeDtypeStruct((G, H), jnp.float32),
        ],
    )(agg2, y2, dinv, b2, bidx)


def _final_body(s_ref, cnt_ref, mx_ref, wp_ref, bp_ref, out_ref):
    s = s_ref[...]
    cnt = cnt_ref[...]
    mean = s / jnp.maximum(cnt, 1.0)
    mx = jnp.where(cnt > 0, mx_ref[...], 0.0)
    out = jnp.dot(s, wp_ref[0:H, :], preferred_element_type=jnp.float32)
    out += jnp.dot(mean, wp_ref[H:2 * H, :], preferred_element_type=jnp.float32)
    out += jnp.dot(mx, wp_ref[2 * H:3 * H, :], preferred_element_type=jnp.float32)
    out_ref[...] = out + bp_ref[...]


def _final_call(s, cnt, mx, Wp, bp):
    return pl.pallas_call(
        _final_body,
        out_shape=jax.ShapeDtypeStruct((G, H), jnp.float32),
    )(s, cnt, mx, Wp, bp)


def kernel(x, edge_index, batch_index, W1, b1, bn_w, bn_b, W2, b2, Wp, bp):
    pad = EP - E
    src = jnp.concatenate([edge_index[0], jnp.zeros((pad,), jnp.int32)])
    pad_dst = N + jnp.arange(pad, dtype=jnp.int32) % (NPAD - N)
    dst = jnp.concatenate([edge_index[1], pad_dst])
    edges2 = jnp.stack([src.reshape(NW * NCH, C),
                        dst.reshape(NW * NCH, C)],
                       axis=1).reshape(2 * NW * NCH, C)
    deg_parts = _deg_sc(edges2)
    src_f = edge_index[0]
    dst_f = edge_index[1]
    y1, dinv = _y1_call(deg_parts, x, W1)
    agg1 = _agg_sc(src_f, dst_f, y1)
    hpre, stats = _hpre_call(agg1, y1, dinv, b1.reshape(1, H))
    y2 = _y2_call(hpre, stats, bn_w.reshape(1, H), bn_b.reshape(1, H), W2, dinv)
    agg2 = _agg_sc(src_f, dst_f, y2)
    s, cnt, mx = _pool_call(agg2, y2, dinv, b2.reshape(1, H),
                            batch_index.reshape(N, 1))
    return _final_call(s, cnt, mx, Wp, bp.reshape(1, H))


# speedup vs baseline: 1.4832x; 1.0011x over previous
import functools

import jax
import jax.numpy as jnp
from jax import lax
from jax.experimental import pallas as pl
from jax.experimental.pallas import tpu as pltpu
from jax.experimental.pallas import tpu_sc as plsc

N = 10000
D = 128
H = 128
E = 320000
G = 64

NC = 2
NS = 16
NW = NC * NS
C = 128
NCH = 80
EP = NW * NCH * C
NPAD = 10240
RPT = NPAD // NS

_mesh = plsc.VectorSubcoreMesh(
    core_axis_name="c", subcore_axis_name="s", num_cores=NC, num_subcores=NS)


BODY = 8
NB = NCH // BODY


@functools.partial(
    pl.kernel,
    out_type=jax.ShapeDtypeStruct((NC, NPAD, H), jnp.float32),
    mesh=_mesh,
    scratch_types=[
        pltpu.VMEM((2 * BODY, C), jnp.int32),
        pltpu.VMEM((C, H), jnp.float32),
        pltpu.VMEM((C, H), jnp.float32),
        pltpu.VMEM_SHARED((NPAD, H), jnp.float32),
        pltpu.SemaphoreType.DMA,
    ],
)
def _deg_sc(edges2_hbm, out_hbm, ei, ones_v, zero_v, deg_sh, sem):
    cid = lax.axis_index("c")
    sid = lax.axis_index("s")
    wid = cid * NS + sid

    def fill(i, _):
        for j in range(H // 16):
            ones_v[i, pl.ds(j * 16, 16)] = jnp.ones((16,), jnp.float32)
            zero_v[i, pl.ds(j * 16, 16)] = jnp.zeros((16,), jnp.float32)
        return 0
    lax.fori_loop(0, C, fill, 0)

    for j in range(RPT // C):
        pltpu.sync_copy(zero_v, deg_sh.at[pl.ds(sid * RPT + j * C, C)])
    plsc.subcore_barrier()

    def body(m, _):
        row0 = pl.multiple_of(2 * (wid * NCH + m * BODY), 8)
        pltpu.sync_copy(edges2_hbm.at[pl.ds(row0, 2 * BODY)], ei)
        ds = [pltpu.async_copy(ones_v, deg_sh.at[ei.at[2 * j + 1]], sem,
                               add=True)
              for j in range(BODY)]
        for j in range(BODY):
            ds[j].wait()
        return 0
    lax.fori_loop(0, NB, body, 0)
    plsc.subcore_barrier()

    pltpu.sync_copy(deg_sh.at[pl.ds(sid * RPT, RPT)],
                    out_hbm.at[cid, pl.ds(sid * RPT, RPT)])


CA = 80
NCHA = E // (NW * CA)


@functools.partial(
    pl.kernel,
    out_type=jax.ShapeDtypeStruct((NC, NPAD, H), jnp.float32),
    mesh=_mesh,
    scratch_types=[
        pltpu.VMEM((CA,), jnp.int32),
        pltpu.VMEM((CA,), jnp.int32),
        pltpu.VMEM((CA, H), jnp.float32),
        pltpu.VMEM((128, H), jnp.float32),
        pltpu.VMEM_SHARED((NPAD, H), jnp.float32),
        pltpu.SemaphoreType.DMA,
    ],
)
def _agg_sc(src_hbm, dst_hbm, y_hbm, out_hbm, src_v, dst_v, rows_v, zero_v,
            agg_sh, sem):
    cid = lax.axis_index("c")
    sid = lax.axis_index("s")

    def fill_zero(i, _):
        for j in range(H // 16):
            zero_v[i, pl.ds(j * 16, 16)] = jnp.zeros((16,), jnp.float32)
        return 0
    lax.fori_loop(0, 128, fill_zero, 0)

    for j in range(RPT // 128):
        pltpu.sync_copy(zero_v, agg_sh.at[pl.ds(sid * RPT + j * 128, 128)])
    plsc.subcore_barrier()

    base = (cid * NS + sid) * (NCHA * CA)

    def body(i, _):
        off = pl.multiple_of(base + i * CA, 8)
        pltpu.sync_copy(src_hbm.at[pl.ds(off, CA)], src_v)
        pltpu.sync_copy(dst_hbm.at[pl.ds(off, CA)], dst_v)
        pltpu.async_copy(y_hbm.at[src_v], rows_v, sem).wait()
        pltpu.sync_copy(rows_v, agg_sh.at[dst_v], add=True)
        return 0
    lax.fori_loop(0, NCHA, body, 0)
    plsc.subcore_barrier()

    pltpu.sync_copy(agg_sh.at[pl.ds(sid * RPT, RPT)],
                    out_hbm.at[cid, pl.ds(sid * RPT, RPT)])


R = 400
RP = 80


def _xw_body(x_ref, w_ref, xw_ref):
    xw_ref[...] = jnp.dot(x_ref[...], w_ref[...],
                          preferred_element_type=jnp.float32)


def _xw_call(x, W1):
    return pl.pallas_call(
        _xw_body,
        grid=(N // R,),
        in_specs=[
            pl.BlockSpec((R, D), lambda i: (i, 0)),
            pl.BlockSpec((D, H), lambda i: (0, 0)),
        ],
        out_specs=pl.BlockSpec((R, H), lambda i: (i, 0)),
        out_shape=jax.ShapeDtypeStruct((N, H), jnp.float32),
    )(x, W1)


def _y1_body(deg_ref, xw_ref, y_ref, dinv_ref):
    deg = deg_ref[0, :, 0:1] + deg_ref[1, :, 0:1] + 1.0
    dinv = lax.rsqrt(deg)
    y_ref[...] = xw_ref[...] * dinv
    dinv_ref[...] = dinv


def _y1_call(deg_parts, xw):
    return pl.pallas_call(
        _y1_body,
        grid=(N // R,),
        in_specs=[
            pl.BlockSpec((NC, R, H), lambda i: (0, i, 0)),
            pl.BlockSpec((R, H), lambda i: (i, 0)),
        ],
        out_specs=[
            pl.BlockSpec((R, H), lambda i: (i, 0)),
            pl.BlockSpec((R, 1), lambda i: (i, 0)),
        ],
        out_shape=[
            jax.ShapeDtypeStruct((N, H), jnp.float32),
            jax.ShapeDtypeStruct((N, 1), jnp.float32),
        ],
    )(deg_parts, xw)


def _hpre_body(agg_ref, y1_ref, dinv_ref, b1_ref, hpre_ref, stats_ref):
    hp = (agg_ref[0] + agg_ref[1] + y1_ref[...]) * dinv_ref[...] + b1_ref[...]
    hpre_ref[...] = hp

    @pl.when(pl.program_id(0) == 0)
    def _():
        stats_ref[...] = jnp.zeros_like(stats_ref)

    stats_ref[0:1, :] += jnp.sum(hp, axis=0, keepdims=True)
    stats_ref[1:2, :] += jnp.sum(hp * hp, axis=0, keepdims=True)


def _hpre_call(agg1, y1, dinv, b1):
    return pl.pallas_call(
        _hpre_body,
        grid=(N // R,),
        in_specs=[
            pl.BlockSpec((NC, R, H), lambda i: (0, i, 0)),
            pl.BlockSpec((R, H), lambda i: (i, 0)),
            pl.BlockSpec((R, 1), lambda i: (i, 0)),
            pl.BlockSpec((1, H), lambda i: (0, 0)),
        ],
        out_specs=[
            pl.BlockSpec((R, H), lambda i: (i, 0)),
            pl.BlockSpec((2, H), lambda i: (0, 0)),
        ],
        out_shape=[
            jax.ShapeDtypeStruct((N, H), jnp.float32),
            jax.ShapeDtypeStruct((2, H), jnp.float32),
        ],
    )(agg1, y1, dinv, b1)


_SELU_SCALE = 1.0507009873554805
_SELU_ALPHA = 1.6732632423543772


def _y2_body(hpre_ref, stats_ref, bnw_ref, bnb_ref, w2_ref, dinv_ref, y2_ref):
    m = stats_ref[0:1, :] * (1.0 / N)
    v = stats_ref[1:2, :] * (1.0 / N) - m * m
    hn = (hpre_ref[...] - m) * lax.rsqrt(v + 1e-5) * bnw_ref[...] + bnb_ref[...]
    act = _SELU_SCALE * jnp.where(hn > 0, hn, _SELU_ALPHA * (jnp.exp(hn) - 1.0))
    y2_ref[...] = jnp.dot(act, w2_ref[...],
                          preferred_element_type=jnp.float32) * dinv_ref[...]


def _y2_call(hpre, stats, bn_w, bn_b, W2, dinv):
    return pl.pallas_call(
        _y2_body,
        grid=(N // R,),
        in_specs=[
            pl.BlockSpec((R, H), lambda i: (i, 0)),
            pl.BlockSpec((2, H), lambda i: (0, 0)),
            pl.BlockSpec((1, H), lambda i: (0, 0)),
            pl.BlockSpec((1, H), lambda i: (0, 0)),
            pl.BlockSpec((H, H), lambda i: (0, 0)),
            pl.BlockSpec((R, 1), lambda i: (i, 0)),
        ],
        out_specs=pl.BlockSpec((R, H), lambda i: (i, 0)),
        out_shape=jax.ShapeDtypeStruct((N, H), jnp.float32),
    )(hpre, stats, bn_w, bn_b, W2, dinv)


def _pool_body(agg_ref, y2_ref, dinv_ref, b2_ref, bidx_ref,
               s_ref, cnt_ref, mx_ref):
    h2 = (agg_ref[0] + agg_ref[1] + y2_ref[...]) * dinv_ref[...] + b2_ref[...]
    cols = lax.broadcasted_iota(jnp.int32, (RP, G), 1)
    mask = bidx_ref[...] == cols
    o = mask.astype(jnp.float32)

    @pl.when(pl.program_id(0) == 0)
    def _():
        s_ref[...] = jnp.zeros_like(s_ref)
        cnt_ref[...] = jnp.zeros_like(cnt_ref)
        mx_ref[...] = jnp.full_like(mx_ref, -jnp.inf)

    dims = (((0,), (0,)), ((), ()))
    s_ref[...] += lax.dot_general(o, h2, dims,
                                  preferred_element_type=jnp.float32)
    cnt_ref[...] += lax.dot_general(o, jnp.ones_like(h2), dims,
                                    preferred_element_type=jnp.float32)
    rows = []
    for g in range(G):
        sel = jnp.where(mask[:, g:g + 1], h2, -jnp.inf)
        rows.append(jnp.max(sel, axis=0, keepdims=True))
    t = jnp.concatenate(rows, axis=0)
    mx_ref[...] = jnp.maximum(mx_ref[...], t)


def _pool_call(agg2, y2, dinv, b2, bidx):
    return pl.pallas_call(
        _pool_body,
        grid=(N // RP,),
        in_specs=[
            pl.BlockSpec((NC, RP, H), lambda i: (0, i, 0)),
            pl.BlockSpec((RP, H), lambda i: (i, 0)),
            pl.BlockSpec((RP, 1), lambda i: (i, 0)),
            pl.BlockSpec((1, H), lambda i: (0, 0)),
            pl.BlockSpec((RP, 1), lambda i: (i, 0)),
        ],
        out_specs=[
            pl.BlockSpec((G, H), lambda i: (0, 0)),
            pl.BlockSpec((G, H), lambda i: (0, 0)),
            pl.BlockSpec((G, H), lambda i: (0, 0)),
        ],
        out_shape=[
            jax.ShapeDtypeStruct((G, H), jnp.float32),
            jax.ShapeDtypeStruct((G, H), jnp.float32),
            jax.ShapeDtypeStruct((G, H), jnp.float32),
        ],
    )(agg2, y2, dinv, b2, bidx)


def _final_body(s_ref, cnt_ref, mx_ref, wp_ref, bp_ref, out_ref):
    s = s_ref[...]
    cnt = cnt_ref[...]
    mean = s / jnp.maximum(cnt, 1.0)
    mx = jnp.where(cnt > 0, mx_ref[...], 0.0)
    out = jnp.dot(s, wp_ref[0:H, :], preferred_element_type=jnp.float32)
    out += jnp.dot(mean, wp_ref[H:2 * H, :], preferred_element_type=jnp.float32)
    out += jnp.dot(mx, wp_ref[2 * H:3 * H, :], preferred_element_type=jnp.float32)
    out_ref[...] = out + bp_ref[...]


def _final_call(s, cnt, mx, Wp, bp):
    return pl.pallas_call(
        _final_body,
        out_shape=jax.ShapeDtypeStruct((G, H), jnp.float32),
    )(s, cnt, mx, Wp, bp)


def kernel(x, edge_index, batch_index, W1, b1, bn_w, bn_b, W2, b2, Wp, bp):
    pad = EP - E
    src = jnp.concatenate([edge_index[0], jnp.zeros((pad,), jnp.int32)])
    pad_dst = N + jnp.arange(pad, dtype=jnp.int32) % (NPAD - N)
    dst = jnp.concatenate([edge_index[1], pad_dst])
    edges2 = jnp.stack([src.reshape(NW * NCH, C),
                        dst.reshape(NW * NCH, C)],
                       axis=1).reshape(2 * NW * NCH, C)
    xw = _xw_call(x, W1)
    deg_parts = _deg_sc(edges2)
    src_f = edge_index[0]
    dst_f = edge_index[1]
    y1, dinv = _y1_call(deg_parts, xw)
    agg1 = _agg_sc(src_f, dst_f, y1)
    hpre, stats = _hpre_call(agg1, y1, dinv, b1.reshape(1, H))
    y2 = _y2_call(hpre, stats, bn_w.reshape(1, H), bn_b.reshape(1, H), W2, dinv)
    agg2 = _agg_sc(src_f, dst_f, y2)
    s, cnt, mx = _pool_call(agg2, y2, dinv, b2.reshape(1, H),
                            batch_index.reshape(N, 1))
    return _final_call(s, cnt, mx, Wp, bp.reshape(1, H))


# speedup vs baseline: 1.4840x; 1.0005x over previous
import functools

import jax
import jax.numpy as jnp
from jax import lax
from jax.experimental import pallas as pl
from jax.experimental.pallas import tpu as pltpu
from jax.experimental.pallas import tpu_sc as plsc

N = 10000
D = 128
H = 128
E = 320000
G = 64

NC = 2
NS = 16
NW = NC * NS
C = 128
NCH = 80
EP = NW * NCH * C
NPAD = 10240
RPT = NPAD // NS

_mesh = plsc.VectorSubcoreMesh(
    core_axis_name="c", subcore_axis_name="s", num_cores=NC, num_subcores=NS)


BODY = 8
NB = NCH // BODY


@functools.partial(
    pl.kernel,
    out_type=jax.ShapeDtypeStruct((NC, NPAD, H), jnp.float32),
    mesh=_mesh,
    scratch_types=[
        pltpu.VMEM((2 * BODY, C), jnp.int32),
        pltpu.VMEM((C, H), jnp.float32),
        pltpu.VMEM((C, H), jnp.float32),
        pltpu.VMEM_SHARED((NPAD, H), jnp.float32),
        pltpu.SemaphoreType.DMA,
    ],
)
def _deg_sc(edges2_hbm, out_hbm, ei, ones_v, zero_v, deg_sh, sem):
    cid = lax.axis_index("c")
    sid = lax.axis_index("s")
    wid = cid * NS + sid

    def fill(i, _):
        for j in range(H // 16):
            ones_v[i, pl.ds(j * 16, 16)] = jnp.ones((16,), jnp.float32)
            zero_v[i, pl.ds(j * 16, 16)] = jnp.zeros((16,), jnp.float32)
        return 0
    lax.fori_loop(0, C, fill, 0)

    for j in range(RPT // C):
        pltpu.sync_copy(zero_v, deg_sh.at[pl.ds(sid * RPT + j * C, C)])
    plsc.subcore_barrier()

    def body(m, _):
        row0 = pl.multiple_of(2 * (wid * NCH + m * BODY), 8)
        pltpu.sync_copy(edges2_hbm.at[pl.ds(row0, 2 * BODY)], ei)
        ds = [pltpu.async_copy(ones_v, deg_sh.at[ei.at[2 * j + 1]], sem,
                               add=True)
              for j in range(BODY)]
        for j in range(BODY):
            ds[j].wait()
        return 0
    lax.fori_loop(0, NB, body, 0)
    plsc.subcore_barrier()

    pltpu.sync_copy(deg_sh.at[pl.ds(sid * RPT, RPT)],
                    out_hbm.at[cid, pl.ds(sid * RPT, RPT)])


CA = 80
NCHA = E // (NW * CA)


@functools.partial(
    pl.kernel,
    out_type=jax.ShapeDtypeStruct((NC, NPAD, H), jnp.float32),
    mesh=_mesh,
    scratch_types=[
        pltpu.VMEM((CA,), jnp.int32),
        pltpu.VMEM((CA,), jnp.int32),
        pltpu.VMEM((CA, H), jnp.float32),
        pltpu.VMEM((128, H), jnp.float32),
        pltpu.VMEM_SHARED((NPAD, H), jnp.float32),
        pltpu.SemaphoreType.DMA,
    ],
)
def _agg_sc(src_hbm, dst_hbm, y_hbm, out_hbm, src_v, dst_v, rows_v, zero_v,
            agg_sh, sem):
    cid = lax.axis_index("c")
    sid = lax.axis_index("s")

    def fill_zero(i, _):
        for j in range(H // 16):
            zero_v[i, pl.ds(j * 16, 16)] = jnp.zeros((16,), jnp.float32)
        return 0
    lax.fori_loop(0, 128, fill_zero, 0)

    for j in range(RPT // 128):
        pltpu.sync_copy(zero_v, agg_sh.at[pl.ds(sid * RPT + j * 128, 128)])
    plsc.subcore_barrier()

    base = (cid * NS + sid) * (NCHA * CA)

    def body(i, _):
        off = pl.multiple_of(base + i * CA, 8)
        pltpu.sync_copy(src_hbm.at[pl.ds(off, CA)], src_v)
        pltpu.sync_copy(dst_hbm.at[pl.ds(off, CA)], dst_v)
        pltpu.async_copy(y_hbm.at[src_v], rows_v, sem).wait()
        pltpu.sync_copy(rows_v, agg_sh.at[dst_v], add=True)
        return 0
    lax.fori_loop(0, NCHA, body, 0)
    plsc.subcore_barrier()

    pltpu.sync_copy(agg_sh.at[pl.ds(sid * RPT, RPT)],
                    out_hbm.at[cid, pl.ds(sid * RPT, RPT)])


R = 400
RP = 80


def _xw_body(x_ref, w_ref, xw_ref):
    xw_ref[...] = jnp.dot(x_ref[...], w_ref[...],
                          preferred_element_type=jnp.float32)


def _xw_call(x, W1):
    return pl.pallas_call(
        _xw_body,
        grid=(N // R,),
        in_specs=[
            pl.BlockSpec((R, D), lambda i: (i, 0)),
            pl.BlockSpec((D, H), lambda i: (0, 0)),
        ],
        out_specs=pl.BlockSpec((R, H), lambda i: (i, 0)),
        out_shape=jax.ShapeDtypeStruct((N, H), jnp.float32),
    )(x, W1)


def _y1_body(deg_ref, xw_ref, y_ref, dinv_ref):
    deg = deg_ref[0, :, 0:1] + deg_ref[1, :, 0:1] + 1.0
    dinv = lax.rsqrt(deg)
    y_ref[...] = xw_ref[...] * dinv
    dinv_ref[...] = dinv


def _y1_call(deg_parts, xw):
    return pl.pallas_call(
        _y1_body,
        grid=(N // R,),
        in_specs=[
            pl.BlockSpec((NC, R, H), lambda i: (0, i, 0)),
            pl.BlockSpec((R, H), lambda i: (i, 0)),
        ],
        out_specs=[
            pl.BlockSpec((R, H), lambda i: (i, 0)),
            pl.BlockSpec((R, 1), lambda i: (i, 0)),
        ],
        out_shape=[
            jax.ShapeDtypeStruct((N, H), jnp.float32),
            jax.ShapeDtypeStruct((N, 1), jnp.float32),
        ],
    )(deg_parts, xw)


def _hpre_body(agg_ref, y1_ref, dinv_ref, b1_ref, hpre_ref, stats_ref):
    hp = (agg_ref[0] + agg_ref[1] + y1_ref[...]) * dinv_ref[...] + b1_ref[...]
    hpre_ref[...] = hp

    @pl.when(pl.program_id(0) == 0)
    def _():
        stats_ref[...] = jnp.zeros_like(stats_ref)

    stats_ref[0:1, :] += jnp.sum(hp, axis=0, keepdims=True)
    stats_ref[1:2, :] += jnp.sum(hp * hp, axis=0, keepdims=True)


def _hpre_call(agg1, y1, dinv, b1):
    return pl.pallas_call(
        _hpre_body,
        grid=(N // R,),
        in_specs=[
            pl.BlockSpec((NC, R, H), lambda i: (0, i, 0)),
            pl.BlockSpec((R, H), lambda i: (i, 0)),
            pl.BlockSpec((R, 1), lambda i: (i, 0)),
            pl.BlockSpec((1, H), lambda i: (0, 0)),
        ],
        out_specs=[
            pl.BlockSpec((R, H), lambda i: (i, 0)),
            pl.BlockSpec((2, H), lambda i: (0, 0)),
        ],
        out_shape=[
            jax.ShapeDtypeStruct((N, H), jnp.float32),
            jax.ShapeDtypeStruct((2, H), jnp.float32),
        ],
    )(agg1, y1, dinv, b1)


_SELU_SCALE = 1.0507009873554805
_SELU_ALPHA = 1.6732632423543772


def _y2_body(hpre_ref, stats_ref, bnw_ref, bnb_ref, w2_ref, dinv_ref, y2_ref):
    m = stats_ref[0:1, :] * (1.0 / N)
    v = stats_ref[1:2, :] * (1.0 / N) - m * m
    hn = (hpre_ref[...] - m) * lax.rsqrt(v + 1e-5) * bnw_ref[...] + bnb_ref[...]
    act = _SELU_SCALE * jnp.where(hn > 0, hn, _SELU_ALPHA * (jnp.exp(hn) - 1.0))
    y2_ref[...] = jnp.dot(act, w2_ref[...],
                          preferred_element_type=jnp.float32) * dinv_ref[...]


def _y2_call(hpre, stats, bn_w, bn_b, W2, dinv):
    return pl.pallas_call(
        _y2_body,
        grid=(N // R,),
        in_specs=[
            pl.BlockSpec((R, H), lambda i: (i, 0)),
            pl.BlockSpec((2, H), lambda i: (0, 0)),
            pl.BlockSpec((1, H), lambda i: (0, 0)),
            pl.BlockSpec((1, H), lambda i: (0, 0)),
            pl.BlockSpec((H, H), lambda i: (0, 0)),
            pl.BlockSpec((R, 1), lambda i: (i, 0)),
        ],
        out_specs=pl.BlockSpec((R, H), lambda i: (i, 0)),
        out_shape=jax.ShapeDtypeStruct((N, H), jnp.float32),
    )(hpre, stats, bn_w, bn_b, W2, dinv)


def _pool_body(agg_ref, y2_ref, dinv_ref, b2_ref, bidx_ref, wp_ref, bp_ref,
               s_ref, cnt_ref, mx_ref, out_ref):
    h2 = (agg_ref[0] + agg_ref[1] + y2_ref[...]) * dinv_ref[...] + b2_ref[...]
    cols = lax.broadcasted_iota(jnp.int32, (RP, G), 1)
    mask = bidx_ref[...] == cols
    o = mask.astype(jnp.float32)

    @pl.when(pl.program_id(0) == 0)
    def _():
        s_ref[...] = jnp.zeros_like(s_ref)
        cnt_ref[...] = jnp.zeros_like(cnt_ref)
        mx_ref[...] = jnp.full_like(mx_ref, -jnp.inf)

    dims = (((0,), (0,)), ((), ()))
    s_ref[...] += lax.dot_general(o, h2, dims,
                                  preferred_element_type=jnp.float32)
    cnt_ref[...] += lax.dot_general(o, jnp.ones_like(h2), dims,
                                    preferred_element_type=jnp.float32)
    rows = []
    for g in range(G):
        sel = jnp.where(mask[:, g:g + 1], h2, -jnp.inf)
        rows.append(jnp.max(sel, axis=0, keepdims=True))
    t = jnp.concatenate(rows, axis=0)
    mx_ref[...] = jnp.maximum(mx_ref[...], t)

    @pl.when(pl.program_id(0) == N // RP - 1)
    def _():
        s = s_ref[...]
        cnt = cnt_ref[...]
        mean = s / jnp.maximum(cnt, 1.0)
        mx = jnp.where(cnt > 0, mx_ref[...], 0.0)
        out = jnp.dot(s, wp_ref[0:H, :], preferred_element_type=jnp.float32)
        out += jnp.dot(mean, wp_ref[H:2 * H, :],
                       preferred_element_type=jnp.float32)
        out += jnp.dot(mx, wp_ref[2 * H:3 * H, :],
                       preferred_element_type=jnp.float32)
        out_ref[...] = out + bp_ref[...]


def _pool_call(agg2, y2, dinv, b2, bidx, Wp, bp):
    return pl.pallas_call(
        _pool_body,
        grid=(N // RP,),
        in_specs=[
            pl.BlockSpec((NC, RP, H), lambda i: (0, i, 0)),
            pl.BlockSpec((RP, H), lambda i: (i, 0)),
            pl.BlockSpec((RP, 1), lambda i: (i, 0)),
            pl.BlockSpec((1, H), lambda i: (0, 0)),
            pl.BlockSpec((RP, 1), lambda i: (i, 0)),
            pl.BlockSpec((3 * H, H), lambda i: (0, 0)),
            pl.BlockSpec((1, H), lambda i: (0, 0)),
        ],
        out_specs=[
            pl.BlockSpec((G, H), lambda i: (0, 0)),
            pl.BlockSpec((G, H), lambda i: (0, 0)),
            pl.BlockSpec((G, H), lambda i: (0, 0)),
            pl.BlockSpec((G, H), lambda i: (0, 0)),
        ],
        out_shape=[
            jax.ShapeDtypeStruct((G, H), jnp.float32),
            jax.ShapeDtypeStruct((G, H), jnp.float32),
            jax.ShapeDtypeStruct((G, H), jnp.float32),
            jax.ShapeDtypeStruct((G, H), jnp.float32),
        ],
    )(agg2, y2, dinv, b2, bidx, Wp, bp)


def kernel(x, edge_index, batch_index, W1, b1, bn_w, bn_b, W2, b2, Wp, bp):
    pad = EP - E
    src = jnp.concatenate([edge_index[0], jnp.zeros((pad,), jnp.int32)])
    pad_dst = N + jnp.arange(pad, dtype=jnp.int32) % (NPAD - N)
    dst = jnp.concatenate([edge_index[1], pad_dst])
    edges2 = jnp.stack([src.reshape(NW * NCH, C),
                        dst.reshape(NW * NCH, C)],
                       axis=1).reshape(2 * NW * NCH, C)
    xw = _xw_call(x, W1)
    deg_parts = _deg_sc(edges2)
    src_f = edge_index[0]
    dst_f = edge_index[1]
    y1, dinv = _y1_call(deg_parts, xw)
    agg1 = _agg_sc(src_f, dst_f, y1)
    hpre, stats = _hpre_call(agg1, y1, dinv, b1.reshape(1, H))
    y2 = _y2_call(hpre, stats, bn_w.reshape(1, H), bn_b.reshape(1, H), W2, dinv)
    agg2 = _agg_sc(src_f, dst_f, y2)
    _, _, _, out = _pool_call(agg2, y2, dinv, b2.reshape(1, H),
                              batch_index.reshape(N, 1), Wp,
                              bp.reshape(1, H))
    return out
